# baseline (speedup 1.0000x reference)
import functools

import jax
import jax.numpy as jnp
from jax import lax
from jax.experimental import pallas as pl
from jax.experimental.pallas import tpu as pltpu
from jax.experimental.pallas import tpu_sc as plsc

_NC = 2
_NS = 16
_NW = _NC * _NS

_K = 16
_NBUF = 3


@functools.partial(jax.jit, static_argnames=("rows_w", "f", "d"))
def _gather_call(x, table, *, rows_w, f, d):
    groups = rows_w // _K
    n_rows = rows_w * _NW
    assert groups >= _NBUF + 1

    mesh = plsc.VectorSubcoreMesh(
        core_axis_name="c", subcore_axis_name="s",
        num_cores=_NC, num_subcores=_NS,
    )

    @functools.partial(
        pl.kernel,
        out_type=jax.ShapeDtypeStruct((n_rows, f, d), jnp.float32),
        mesh=mesh,
        scratch_types=[
            pltpu.VMEM((rows_w, f), jnp.int32),
            pltpu.VMEM((_NBUF, _K, f, d), jnp.float32),
            pltpu.SemaphoreType.DMA((_NBUF,)),
            pltpu.SemaphoreType.DMA((_NBUF,)),
        ],
        compiler_params=pltpu.CompilerParams(use_tc_tiling_on_sc=False),
    )
    def body(idx_hbm, table_hbm, out_hbm, sx, rows_v, gsem, wsem):
        wid = lax.axis_index("s") * _NC + lax.axis_index("c")
        base = wid * rows_w
        pltpu.sync_copy(idx_hbm.at[pl.ds(base, rows_w)], sx)

        def fire_g(g, b):
            for j in range(_K):
                pltpu.async_copy(
                    table_hbm.at[sx.at[g * _K + j]],
                    rows_v.at[b, j],
                    gsem.at[b])

        def drain_g(b):
            pltpu.make_async_copy(
                out_hbm.at[pl.ds(0, _K)], rows_v.at[b], gsem.at[b]).wait()

        def fire_w(g, b):
            pltpu.async_copy(rows_v.at[b],
                             out_hbm.at[pl.ds(base + g * _K, _K)],
                             wsem.at[b])

        def wait_w(b):
            pltpu.make_async_copy(rows_v.at[b],
                                  out_hbm.at[pl.ds(base, _K)],
                                  wsem.at[b]).wait()

        fire_g(0, 0)
        fire_g(1, 1)
        drain_g(0)
        fire_w(0, 0)
        fire_g(2, 2)
        drain_g(1)
        fire_w(1, 1)

        def step(g, _):
            b = g % _NBUF
            bn = (g + 1) % _NBUF
            wait_w(bn)
            fire_g(g + 1, bn)
            drain_g(b)
            fire_w(g, b)
            return ()

        lax.fori_loop(2, groups - 1, step, (), unroll=False)

        g = groups - 1
        wait_w((g + 1) % _NBUF)
        drain_g(g % _NBUF)
        fire_w(g, g % _NBUF)
        wait_w((groups - 2) % _NBUF)
        wait_w((groups - 1) % _NBUF)

    return body(x, table)


def kernel(x, table):
    n_rows, f = x.shape
    d = table.shape[1]
    assert n_rows % (_NW * _K) == 0
    rows_w = n_rows // _NW
    return _gather_call(x.astype(jnp.int32), table, rows_w=rows_w, f=f, d=d)


# speedup vs baseline: 1.0741x; 1.0741x over previous
import functools

import jax
import jax.numpy as jnp
from jax import lax
from jax.experimental import pallas as pl
from jax.experimental.pallas import tpu as pltpu
from jax.experimental.pallas import tpu_sc as plsc

_NC = 2
_NS = 16
_NW = _NC * _NS

_K = 16
_NBUF = 3


@functools.partial(jax.jit, static_argnames=("rows_w", "f", "d"))
def _gather_call(x, table, *, rows_w, f, d):
    groups = rows_w // _K
    n_rows = rows_w * _NW
    assert groups >= _NBUF + 1

    mesh = plsc.VectorSubcoreMesh(
        core_axis_name="c", subcore_axis_name="s",
        num_cores=_NC, num_subcores=_NS,
    )

    @functools.partial(
        pl.kernel,
        out_type=jax.ShapeDtypeStruct((n_rows, f, d), jnp.float32),
        mesh=mesh,
        scratch_types=[
            pltpu.VMEM((rows_w, f), jnp.int32),
            pltpu.VMEM((_NBUF, _K, f, d), jnp.float32),
            pltpu.SemaphoreType.DMA((_NBUF,)),
            pltpu.SemaphoreType.DMA((_NBUF,)),
        ],
        compiler_params=pltpu.CompilerParams(use_tc_tiling_on_sc=False),
    )
    def body(idx_hbm, table_hbm, out_hbm, sx, rows_v, gsem, wsem):
        wid = lax.axis_index("s") * _NC + lax.axis_index("c")
        base = wid * rows_w
        pltpu.sync_copy(idx_hbm.at[pl.ds(base, rows_w)], sx)

        def fire_g(g, b):
            for j in range(_K):
                pltpu.async_copy(
                    table_hbm.at[sx.at[g * _K + j]],
                    rows_v.at[b, j],
                    gsem.at[b])

        def drain_g(b):
            pltpu.make_async_copy(
                out_hbm.at[pl.ds(0, _K)], rows_v.at[b], gsem.at[b]).wait()

        def fire_w(g, b):
            pltpu.async_copy(rows_v.at[b],
                             out_hbm.at[pl.ds(base + g * _K, _K)],
                             wsem.at[b])

        def wait_w(b):
            pltpu.make_async_copy(rows_v.at[b],
                                  out_hbm.at[pl.ds(base, _K)],
                                  wsem.at[b]).wait()

        fire_g(0, 0)
        fire_g(1, 1)
        drain_g(0)
        fire_w(0, 0)
        fire_g(2, 2)
        drain_g(1)
        fire_w(1, 1)

        def step(g, _):
            b = g % _NBUF
            bn = (g + 1) % _NBUF
            wait_w(bn)
            fire_g(g + 1, bn)
            drain_g(b)
            fire_w(g, b)
            return ()

        lax.fori_loop(2, groups - 1, step, (), unroll=False)

        g = groups - 1
        wait_w((g + 1) % _NBUF)
        drain_g(g % _NBUF)
        fire_w(g, g % _NBUF)
        wait_w((groups - 2) % _NBUF)
        wait_w((groups - 1) % _NBUF)

    return body(x, table)


def kernel(x, table):
    n_rows, f = x.shape
    v, d = table.shape
    assert n_rows % (_NW * _K) == 0
    rows_w = n_rows // _NW
    tpad = jnp.pad(table, ((0, 0), (0, 128 - d))).reshape(2 * v, d)
    x2 = x.astype(jnp.int32) * 2
    return _gather_call(x2, tpad, rows_w=rows_w, f=f, d=d)


# speedup vs baseline: 1.1167x; 1.0397x over previous
import functools

import jax
import jax.numpy as jnp
from jax import lax
from jax.experimental import pallas as pl
from jax.experimental.pallas import tpu as pltpu
from jax.experimental.pallas import tpu_sc as plsc

_NC = 2
_NS = 16
_NW = _NC * _NS

_K = 16
_NBUF = 3


@functools.partial(jax.jit, static_argnames=("rows_w", "f", "d"))
def _gather_call(x, table, *, rows_w, f, d):
    groups = rows_w // _K
    n_rows = rows_w * _NW
    assert groups >= _NBUF + 1

    mesh = plsc.VectorSubcoreMesh(
        core_axis_name="c", subcore_axis_name="s",
        num_cores=_NC, num_subcores=_NS,
    )

    @functools.partial(
        pl.kernel,
        out_type=jax.ShapeDtypeStruct((f, n_rows, d), jnp.float32),
        mesh=mesh,
        scratch_types=[
            pltpu.VMEM((rows_w, f), jnp.int32),
            pltpu.VMEM((_NBUF, _K, f, d), jnp.float32),
            pltpu.SemaphoreType.DMA((_NBUF,)),
            pltpu.SemaphoreType.DMA((_NBUF,)),
        ],
        compiler_params=pltpu.CompilerParams(use_tc_tiling_on_sc=False),
    )
    def body(idx_hbm, table_hbm, out_hbm, sx, rows_v, gsem, wsem):
        wid = lax.axis_index("s") * _NC + lax.axis_index("c")
        base = wid * rows_w
        pltpu.sync_copy(idx_hbm.at[pl.ds(base, rows_w)], sx)

        def fire_g(g, b):
            for j in range(_K):
                pltpu.async_copy(
                    table_hbm.at[sx.at[g * _K + j]],
                    rows_v.at[b, j],
                    gsem.at[b])

        def drain_g(b):
            for j in range(_K):
                pltpu.make_async_copy(
                    table_hbm.at[sx.at[j]],
                    rows_v.at[b, j],
                    gsem.at[b]).wait()

        def fire_w(g, b):
            for j in range(f):
                pltpu.async_copy(rows_v.at[b, :, j],
                                 out_hbm.at[j, pl.ds(base + g * _K, _K)],
                                 wsem.at[b])

        def wait_w(b):
            for j in range(f):
                pltpu.make_async_copy(rows_v.at[b, :, j],
                                      out_hbm.at[j, pl.ds(base, _K)],
                                      wsem.at[b]).wait()

        fire_g(0, 0)
        fire_g(1, 1)
        drain_g(0)
        fire_w(0, 0)
        fire_g(2, 2)
        drain_g(1)
        fire_w(1, 1)

        def step(g, _):
            b = g % _NBUF
            bn = (g + 1) % _NBUF
            wait_w(bn)
            fire_g(g + 1, bn)
            drain_g(b)
            fire_w(g, b)
            return ()

        lax.fori_loop(2, groups - 1, step, (), unroll=False)

        g = groups - 1
        wait_w((g + 1) % _NBUF)
        drain_g(g % _NBUF)
        fire_w(g, g % _NBUF)
        wait_w((groups - 2) % _NBUF)
        wait_w((groups - 1) % _NBUF)

    return body(x, table)


def kernel(x, table):
    n_rows, f = x.shape
    v, d = table.shape
    assert n_rows % (_NW * _K) == 0
    rows_w = n_rows // _NW
    tpad = jnp.pad(table, ((0, 0), (0, 128 - d))).reshape(2 * v, d)
    x2 = x.astype(jnp.int32) * 2
    out_t = _gather_call(x2, tpad, rows_w=rows_w, f=f, d=d)
    return jnp.transpose(out_t, (1, 0, 2))
